# baseline (device time: 2306375 ns/iter reference)
import jax
import jax.numpy as jnp
from jax import lax
from jax.experimental import pallas as pl
from jax.experimental.pallas import tpu as pltpu

T_LOC = 1024
D = 1024
E_LOC = 8
E_GLB = 16
F = 4096
C = 320
FT = 1024


def _peer():
    return (lax.axis_index("x"), 1 - lax.axis_index("y"))


def _peer_barrier():
    bar = pltpu.get_barrier_semaphore()
    pl.semaphore_signal(
        bar, inc=1, device_id=_peer(), device_id_type=pl.DeviceIdType.MESH
    )
    pl.semaphore_wait(bar, 1)


def _ag_x_router(x, router):

    def body(x_ref, r_ref, xf_ref, rs_ref, send_sems, recv_sems):
        my_y = lax.axis_index("y")
        _peer_barrier()
        row0 = my_y * T_LOC
        xf_ref[pl.ds(row0, T_LOC), :] = x_ref[...].astype(jnp.bfloat16)
        rs_ref[my_y] = r_ref[...]
        rdma_x = pltpu.make_async_remote_copy(
            src_ref=xf_ref.at[pl.ds(row0, T_LOC), :],
            dst_ref=xf_ref.at[pl.ds(row0, T_LOC), :],
            send_sem=send_sems.at[0],
            recv_sem=recv_sems.at[0],
            device_id=_peer(),
            device_id_type=pl.DeviceIdType.MESH,
        )
        rdma_r = pltpu.make_async_remote_copy(
            src_ref=rs_ref.at[my_y],
            dst_ref=rs_ref.at[my_y],
            send_sem=send_sems.at[1],
            recv_sem=recv_sems.at[1],
            device_id=_peer(),
            device_id_type=pl.DeviceIdType.MESH,
        )
        rdma_x.start()
        rdma_r.start()
        rdma_r.wait()
        rdma_x.wait()

    return pl.pallas_call(
        body,
        out_shape=(
            jax.ShapeDtypeStruct((2 * T_LOC, D), jnp.bfloat16),
            jax.ShapeDtypeStruct((2, D, E_LOC), jnp.float32),
        ),
        in_specs=[
            pl.BlockSpec(memory_space=pltpu.VMEM),
            pl.BlockSpec(memory_space=pltpu.VMEM),
        ],
        out_specs=(
            pl.BlockSpec(memory_space=pltpu.VMEM),
            pl.BlockSpec(memory_space=pltpu.VMEM),
        ),
        scratch_shapes=[
            pltpu.SemaphoreType.DMA((2,)),
            pltpu.SemaphoreType.DMA((2,)),
        ],
        compiler_params=pltpu.CompilerParams(collective_id=0),
    )(x, router)


def _ag_gates(gates_mine):

    def body(g_ref, gf_ref, send_sem, recv_sem):
        my_y = lax.axis_index("y")
        _peer_barrier()
        row0 = my_y * T_LOC
        gf_ref[pl.ds(row0, T_LOC), :] = g_ref[...]
        rdma = pltpu.make_async_remote_copy(
            src_ref=gf_ref.at[pl.ds(row0, T_LOC), :],
            dst_ref=gf_ref.at[pl.ds(row0, T_LOC), :],
            send_sem=send_sem,
            recv_sem=recv_sem,
            device_id=_peer(),
            device_id_type=pl.DeviceIdType.MESH,
        )
        rdma.start()
        rdma.wait()

    return pl.pallas_call(
        body,
        out_shape=jax.ShapeDtypeStruct((2 * T_LOC, E_GLB), jnp.float32),
        in_specs=[pl.BlockSpec(memory_space=pltpu.VMEM)],
        out_specs=pl.BlockSpec(memory_space=pltpu.VMEM),
        scratch_shapes=[pltpu.SemaphoreType.DMA, pltpu.SemaphoreType.DMA],
        compiler_params=pltpu.CompilerParams(collective_id=1),
    )(gates_mine)


def _expert_ffn(xg, W1, W2):

    def body(xg_ref, w1_ref, w2_ref, out_ref):
        fi = pl.program_id(1)
        xe = xg_ref[0]
        w1 = w1_ref[0].astype(jnp.bfloat16)
        h = jnp.dot(xe, w1, preferred_element_type=jnp.float32)
        h = jnp.maximum(h, 0.0).astype(jnp.bfloat16)
        w2 = w2_ref[0].astype(jnp.bfloat16)
        y = jnp.dot(h, w2, preferred_element_type=jnp.float32)

        @pl.when(fi == 0)
        def _():
            out_ref[...] = jnp.zeros_like(out_ref)

        out_ref[...] += y[None]

    return pl.pallas_call(
        body,
        grid=(E_LOC, F // FT),
        in_specs=[
            pl.BlockSpec((1, C, D), lambda e, fi: (e, 0, 0)),
            pl.BlockSpec((1, D, FT), lambda e, fi: (e, 0, fi)),
            pl.BlockSpec((1, FT, D), lambda e, fi: (e, fi, 0)),
        ],
        out_specs=pl.BlockSpec((1, C, D), lambda e, fi: (e, 0, 0)),
        out_shape=jax.ShapeDtypeStruct((E_LOC, C, D), jnp.float32),
        compiler_params=pltpu.CompilerParams(
            dimension_semantics=("parallel", "arbitrary")
        ),
    )(xg, W1, W2)


def _reduce_scatter(out_full):

    def body(of_ref, out_ref, sbuf, rbuf, send_sem, recv_sem):
        my_y = lax.axis_index("y")
        _peer_barrier()
        sbuf[...] = of_ref[pl.ds((1 - my_y) * T_LOC, T_LOC), :].astype(
            jnp.bfloat16
        )
        rdma = pltpu.make_async_remote_copy(
            src_ref=sbuf,
            dst_ref=rbuf,
            send_sem=send_sem,
            recv_sem=recv_sem,
            device_id=_peer(),
            device_id_type=pl.DeviceIdType.MESH,
        )
        rdma.start()
        rdma.wait()
        out_ref[...] = of_ref[pl.ds(my_y * T_LOC, T_LOC), :] + rbuf[
            ...
        ].astype(jnp.float32)

    return pl.pallas_call(
        body,
        out_shape=jax.ShapeDtypeStruct((T_LOC, D), jnp.float32),
        in_specs=[pl.BlockSpec(memory_space=pltpu.VMEM)],
        out_specs=pl.BlockSpec(memory_space=pltpu.VMEM),
        scratch_shapes=[
            pltpu.VMEM((T_LOC, D), jnp.bfloat16),
            pltpu.VMEM((T_LOC, D), jnp.bfloat16),
            pltpu.SemaphoreType.DMA,
            pltpu.SemaphoreType.DMA,
        ],
        compiler_params=pltpu.CompilerParams(collective_id=2),
    )(out_full)


def kernel(x, router, W1, W2):
    my_y = lax.axis_index("y")

    x_full, r_stack = _ag_x_router(x, router)
    router_full = jnp.moveaxis(r_stack, 0, 1).reshape(D, E_GLB)

    gates_mine = jnp.dot(x, router_full, precision=lax.Precision.HIGHEST)
    gates_full = _ag_gates(gates_mine)

    TT = 2 * T_LOC
    topv, topi = lax.top_k(gates_full, 2)
    w = jax.nn.softmax(topv, axis=-1)
    le = topi - my_y * E_LOC
    valid = (le >= 0) & (le < E_LOC)
    t_ids = jnp.broadcast_to(
        jnp.arange(TT, dtype=jnp.int32)[:, None], (TT, 2)
    )
    big = jnp.int32(E_LOC * TT + TT)
    key = jnp.where(valid, le * TT + t_ids, big).reshape(-1)
    _, st = lax.sort((key, t_ids.reshape(-1)), num_keys=1)

    onehot = (
        le[:, :, None] == jnp.arange(E_LOC)[None, None, :]
    ) & valid[:, :, None]
    loads = onehot.sum(axis=(0, 1)).astype(jnp.int32)
    estart = jnp.cumsum(loads) - loads
    c_idx = jnp.arange(C, dtype=jnp.int32)[None, :]
    jpos = jnp.where(c_idx < loads[:, None], estart[:, None] + c_idx, 0)
    idx = st[jpos.reshape(-1)]

    xg = x_full[idx].reshape(E_LOC, C, D)
    yg = _expert_ffn(xg, W1, W2)

    mask = onehot.any(axis=1)
    pos = jnp.cumsum(mask.astype(jnp.int32), axis=0) - 1
    le_c = jnp.clip(le, 0, E_LOC - 1)
    pos_k = jnp.take_along_axis(pos, le_c, axis=1)
    cvalid = valid & (pos_k < C)
    ck = jnp.where(cvalid, le_c * C + pos_k, 0)
    wk = jnp.where(cvalid, w, 0.0)
    yf = yg.reshape(E_LOC * C, D)
    out_full = yf[ck[:, 0]] * wk[:, 0:1] + yf[ck[:, 1]] * wk[:, 1:2]

    return _reduce_scatter(out_full)


# device time: 220245 ns/iter; 10.4719x vs baseline; 10.4719x over previous
import jax
import jax.numpy as jnp
from jax import lax
from jax.experimental import pallas as pl
from jax.experimental.pallas import tpu as pltpu

T_LOC = 1024
D = 1024
E_LOC = 8
E_GLB = 16
F = 4096
C = 320
FT = 1024


def _peer():
    return (lax.axis_index("x"), 1 - lax.axis_index("y"))


def _peer_barrier():
    bar = pltpu.get_barrier_semaphore()
    pl.semaphore_signal(
        bar, inc=1, device_id=_peer(), device_id_type=pl.DeviceIdType.MESH
    )
    pl.semaphore_wait(bar, 1)


def _ag_x_router(x, router):

    def body(x_ref, r_ref, xf_ref, rs_ref, send_sems, recv_sems):
        my_y = lax.axis_index("y")
        _peer_barrier()
        row0 = my_y * T_LOC
        xf_ref[pl.ds(row0, T_LOC), :] = x_ref[...].astype(jnp.bfloat16)
        rs_ref[my_y] = r_ref[...]
        rdma_x = pltpu.make_async_remote_copy(
            src_ref=xf_ref.at[pl.ds(row0, T_LOC), :],
            dst_ref=xf_ref.at[pl.ds(row0, T_LOC), :],
            send_sem=send_sems.at[0],
            recv_sem=recv_sems.at[0],
            device_id=_peer(),
            device_id_type=pl.DeviceIdType.MESH,
        )
        rdma_r = pltpu.make_async_remote_copy(
            src_ref=rs_ref.at[my_y],
            dst_ref=rs_ref.at[my_y],
            send_sem=send_sems.at[1],
            recv_sem=recv_sems.at[1],
            device_id=_peer(),
            device_id_type=pl.DeviceIdType.MESH,
        )
        rdma_x.start()
        rdma_r.start()
        rdma_r.wait()
        rdma_x.wait()

    return pl.pallas_call(
        body,
        out_shape=(
            jax.ShapeDtypeStruct((2 * T_LOC, D), jnp.bfloat16),
            jax.ShapeDtypeStruct((2, D, E_LOC), jnp.float32),
        ),
        in_specs=[
            pl.BlockSpec(memory_space=pltpu.VMEM),
            pl.BlockSpec(memory_space=pltpu.VMEM),
        ],
        out_specs=(
            pl.BlockSpec(memory_space=pltpu.VMEM),
            pl.BlockSpec(memory_space=pltpu.VMEM),
        ),
        scratch_shapes=[
            pltpu.SemaphoreType.DMA((2,)),
            pltpu.SemaphoreType.DMA((2,)),
        ],
        compiler_params=pltpu.CompilerParams(collective_id=0),
    )(x, router)


def _ag_gates(gates_mine):

    def body(g_ref, gf_ref, send_sem, recv_sem):
        my_y = lax.axis_index("y")
        _peer_barrier()
        row0 = my_y * T_LOC
        gf_ref[pl.ds(row0, T_LOC), :] = g_ref[...]
        rdma = pltpu.make_async_remote_copy(
            src_ref=gf_ref.at[pl.ds(row0, T_LOC), :],
            dst_ref=gf_ref.at[pl.ds(row0, T_LOC), :],
            send_sem=send_sem,
            recv_sem=recv_sem,
            device_id=_peer(),
            device_id_type=pl.DeviceIdType.MESH,
        )
        rdma.start()
        rdma.wait()

    return pl.pallas_call(
        body,
        out_shape=jax.ShapeDtypeStruct((2 * T_LOC, E_GLB), jnp.float32),
        in_specs=[pl.BlockSpec(memory_space=pltpu.VMEM)],
        out_specs=pl.BlockSpec(memory_space=pltpu.VMEM),
        scratch_shapes=[pltpu.SemaphoreType.DMA, pltpu.SemaphoreType.DMA],
        compiler_params=pltpu.CompilerParams(collective_id=1),
    )(gates_mine)


def _expert_ffn(x_full, q, W1, W2):

    def body(xf_ref, q_ref, w1_ref, w2_ref, out_ref, xg_ref):
        fi = pl.program_id(1)

        @pl.when(fi == 0)
        def _():
            xg_ref[...] = jnp.dot(
                q_ref[0], xf_ref[...], preferred_element_type=jnp.float32
            ).astype(jnp.bfloat16)

        w1 = w1_ref[0].astype(jnp.bfloat16)
        h = jnp.dot(xg_ref[...], w1, preferred_element_type=jnp.float32)
        h = jnp.maximum(h, 0.0).astype(jnp.bfloat16)
        w2 = w2_ref[0].astype(jnp.bfloat16)
        y = jnp.dot(h, w2, preferred_element_type=jnp.float32)

        @pl.when(fi == 0)
        def _():
            out_ref[...] = jnp.zeros_like(out_ref)

        out_ref[...] += y[None]

    return pl.pallas_call(
        body,
        grid=(E_LOC, F // FT),
        in_specs=[
            pl.BlockSpec((2 * T_LOC, D), lambda e, fi: (0, 0)),
            pl.BlockSpec((1, C, 2 * T_LOC), lambda e, fi: (e, 0, 0)),
            pl.BlockSpec((1, D, FT), lambda e, fi: (e, 0, fi)),
            pl.BlockSpec((1, FT, D), lambda e, fi: (e, fi, 0)),
        ],
        out_specs=pl.BlockSpec((1, C, D), lambda e, fi: (e, 0, 0)),
        out_shape=jax.ShapeDtypeStruct((E_LOC, C, D), jnp.float32),
        scratch_shapes=[pltpu.VMEM((C, D), jnp.bfloat16)],
        compiler_params=pltpu.CompilerParams(
            dimension_semantics=("arbitrary", "arbitrary")
        ),
    )(x_full, q, W1, W2)


def _combine_reduce_scatter(p, yf):

    def body(p_ref, yf_ref, out_ref, sbuf, rbuf, send_sem, recv_sem):
        my_y = lax.axis_index("y")
        _peer_barrier()
        yb = yf_ref[...].astype(jnp.bfloat16)
        sbuf[...] = jnp.dot(
            p_ref[pl.ds((1 - my_y) * T_LOC, T_LOC), :],
            yb,
            preferred_element_type=jnp.float32,
        ).astype(jnp.bfloat16)
        rdma = pltpu.make_async_remote_copy(
            src_ref=sbuf,
            dst_ref=rbuf,
            send_sem=send_sem,
            recv_sem=recv_sem,
            device_id=_peer(),
            device_id_type=pl.DeviceIdType.MESH,
        )
        rdma.start()
        out_ref[...] = jnp.dot(
            p_ref[pl.ds(my_y * T_LOC, T_LOC), :],
            yb,
            preferred_element_type=jnp.float32,
        )
        rdma.wait()
        out_ref[...] += rbuf[...].astype(jnp.float32)

    return pl.pallas_call(
        body,
        out_shape=jax.ShapeDtypeStruct((T_LOC, D), jnp.float32),
        in_specs=[
            pl.BlockSpec(memory_space=pltpu.VMEM),
            pl.BlockSpec(memory_space=pltpu.VMEM),
        ],
        out_specs=pl.BlockSpec(memory_space=pltpu.VMEM),
        scratch_shapes=[
            pltpu.VMEM((T_LOC, D), jnp.bfloat16),
            pltpu.VMEM((T_LOC, D), jnp.bfloat16),
            pltpu.SemaphoreType.DMA,
            pltpu.SemaphoreType.DMA,
        ],
        compiler_params=pltpu.CompilerParams(collective_id=2),
    )(p, yf)


def kernel(x, router, W1, W2):
    my_y = lax.axis_index("y")

    x_full, r_stack = _ag_x_router(x, router)
    router_full = jnp.moveaxis(r_stack, 0, 1).reshape(D, E_GLB)

    gates_mine = jnp.dot(x, router_full, precision=lax.Precision.HIGHEST)
    gates_full = _ag_gates(gates_mine)

    TT = 2 * T_LOC
    topv, topi = lax.top_k(gates_full, 2)
    w = jax.nn.softmax(topv, axis=-1)
    le = topi - my_y * E_LOC
    valid = (le >= 0) & (le < E_LOC)
    t_ids = jnp.broadcast_to(
        jnp.arange(TT, dtype=jnp.int32)[:, None], (TT, 2)
    )
    big = jnp.int32(E_LOC * TT + TT)
    key = jnp.where(valid, le * TT + t_ids, big).reshape(-1)
    _, st = lax.sort((key, t_ids.reshape(-1)), num_keys=1)

    onehot = (
        le[:, :, None] == jnp.arange(E_LOC)[None, None, :]
    ) & valid[:, :, None]
    loads = onehot.sum(axis=(0, 1)).astype(jnp.int32)
    estart = jnp.cumsum(loads) - loads
    c_idx = jnp.arange(C, dtype=jnp.int32)[None, :]
    jpos = jnp.where(c_idx < loads[:, None], estart[:, None] + c_idx, 0)
    idx = st[jpos.reshape(-1)]

    q = (
        idx[:, None] == jnp.arange(TT, dtype=jnp.int32)[None, :]
    ).astype(jnp.bfloat16).reshape(E_LOC, C, TT)

    yg = _expert_ffn(x_full, q, W1, W2)

    mask = onehot.any(axis=1)
    pos = jnp.cumsum(mask.astype(jnp.int32), axis=0) - 1
    le_c = jnp.clip(le, 0, E_LOC - 1)
    pos_k = jnp.sum(
        pos[:, None, :] * onehot.astype(jnp.int32), axis=2
    )
    cvalid = valid & (pos_k < C)
    ck = jnp.where(cvalid, le_c * C + pos_k, 0)
    wk = jnp.where(cvalid, w, 0.0)
    slot_iota = jnp.arange(E_LOC * C, dtype=jnp.int32)[None, :]
    p = (
        (ck[:, 0:1] == slot_iota) * wk[:, 0:1]
        + (ck[:, 1:2] == slot_iota) * wk[:, 1:2]
    ).astype(jnp.bfloat16)

    return _combine_reduce_scatter(p, yg.reshape(E_LOC * C, D))


# device time: 207740 ns/iter; 11.1022x vs baseline; 1.0602x over previous
import jax
import jax.numpy as jnp
from jax import lax
from jax.experimental import pallas as pl
from jax.experimental.pallas import tpu as pltpu

T_LOC = 1024
D = 1024
E_LOC = 8
E_GLB = 16
F = 4096
C = 320
FT = 1024


def _peer():
    return (lax.axis_index("x"), 1 - lax.axis_index("y"))


def _peer_barrier():
    bar = pltpu.get_barrier_semaphore()
    pl.semaphore_signal(
        bar, inc=1, device_id=_peer(), device_id_type=pl.DeviceIdType.MESH
    )
    pl.semaphore_wait(bar, 1)


def _ag_x_router(x, router):

    def body(x_ref, r_ref, xf_ref, rs_ref, send_sems, recv_sems):
        my_y = lax.axis_index("y")
        _peer_barrier()
        row0 = my_y * T_LOC
        xf_ref[pl.ds(row0, T_LOC), :] = x_ref[...].astype(jnp.bfloat16)
        rs_ref[my_y] = r_ref[...]
        rdma_x = pltpu.make_async_remote_copy(
            src_ref=xf_ref.at[pl.ds(row0, T_LOC), :],
            dst_ref=xf_ref.at[pl.ds(row0, T_LOC), :],
            send_sem=send_sems.at[0],
            recv_sem=recv_sems.at[0],
            device_id=_peer(),
            device_id_type=pl.DeviceIdType.MESH,
        )
        rdma_r = pltpu.make_async_remote_copy(
            src_ref=rs_ref.at[my_y],
            dst_ref=rs_ref.at[my_y],
            send_sem=send_sems.at[1],
            recv_sem=recv_sems.at[1],
            device_id=_peer(),
            device_id_type=pl.DeviceIdType.MESH,
        )
        rdma_x.start()
        rdma_r.start()
        rdma_r.wait()
        rdma_x.wait()

    return pl.pallas_call(
        body,
        out_shape=(
            jax.ShapeDtypeStruct((2 * T_LOC, D), jnp.bfloat16),
            jax.ShapeDtypeStruct((2, D, E_LOC), jnp.float32),
        ),
        in_specs=[
            pl.BlockSpec(memory_space=pltpu.VMEM),
            pl.BlockSpec(memory_space=pltpu.VMEM),
        ],
        out_specs=(
            pl.BlockSpec(memory_space=pltpu.VMEM),
            pl.BlockSpec(memory_space=pltpu.VMEM),
        ),
        scratch_shapes=[
            pltpu.SemaphoreType.DMA((2,)),
            pltpu.SemaphoreType.DMA((2,)),
        ],
        compiler_params=pltpu.CompilerParams(collective_id=0),
    )(x, router)


def _ag_gates(gates_mine):

    def body(g_ref, gf_ref, send_sem, recv_sem):
        my_y = lax.axis_index("y")
        _peer_barrier()
        row0 = my_y * T_LOC
        gf_ref[pl.ds(row0, T_LOC), :] = g_ref[...]
        rdma = pltpu.make_async_remote_copy(
            src_ref=gf_ref.at[pl.ds(row0, T_LOC), :],
            dst_ref=gf_ref.at[pl.ds(row0, T_LOC), :],
            send_sem=send_sem,
            recv_sem=recv_sem,
            device_id=_peer(),
            device_id_type=pl.DeviceIdType.MESH,
        )
        rdma.start()
        rdma.wait()

    return pl.pallas_call(
        body,
        out_shape=jax.ShapeDtypeStruct((2 * T_LOC, E_GLB), jnp.float32),
        in_specs=[pl.BlockSpec(memory_space=pltpu.VMEM)],
        out_specs=pl.BlockSpec(memory_space=pltpu.VMEM),
        scratch_shapes=[pltpu.SemaphoreType.DMA, pltpu.SemaphoreType.DMA],
        compiler_params=pltpu.CompilerParams(collective_id=1),
    )(gates_mine)


def _expert_ffn(x_full, q, W1, W2):

    def body(xf_ref, q_ref, w1_ref, w2_ref, out_ref, xg_ref):
        fi = pl.program_id(1)

        @pl.when(fi == 0)
        def _():
            xg_ref[...] = jnp.dot(
                q_ref[0], xf_ref[...], preferred_element_type=jnp.float32
            ).astype(jnp.bfloat16)

        w1 = w1_ref[0].astype(jnp.bfloat16)
        h = jnp.dot(xg_ref[...], w1, preferred_element_type=jnp.float32)
        h = jnp.maximum(h, 0.0).astype(jnp.bfloat16)
        w2 = w2_ref[0].astype(jnp.bfloat16)
        y = jnp.dot(h, w2, preferred_element_type=jnp.float32)

        @pl.when(fi == 0)
        def _():
            out_ref[...] = jnp.zeros_like(out_ref)

        out_ref[...] += y[None]

    return pl.pallas_call(
        body,
        grid=(E_LOC, F // FT),
        in_specs=[
            pl.BlockSpec((2 * T_LOC, D), lambda e, fi: (0, 0)),
            pl.BlockSpec((1, C, 2 * T_LOC), lambda e, fi: (e, 0, 0)),
            pl.BlockSpec((1, D, FT), lambda e, fi: (e, 0, fi)),
            pl.BlockSpec((1, FT, D), lambda e, fi: (e, fi, 0)),
        ],
        out_specs=pl.BlockSpec((1, C, D), lambda e, fi: (e, 0, 0)),
        out_shape=jax.ShapeDtypeStruct((E_LOC, C, D), jnp.float32),
        scratch_shapes=[pltpu.VMEM((C, D), jnp.bfloat16)],
        compiler_params=pltpu.CompilerParams(
            dimension_semantics=("arbitrary", "arbitrary")
        ),
    )(x_full, q, W1, W2)


def _combine_reduce_scatter(p, yf):

    def body(p_ref, yf_ref, out_ref, sbuf, rbuf, send_sem, recv_sem):
        my_y = lax.axis_index("y")
        _peer_barrier()
        yb = yf_ref[...].astype(jnp.bfloat16)
        sbuf[...] = jnp.dot(
            p_ref[pl.ds((1 - my_y) * T_LOC, T_LOC), :],
            yb,
            preferred_element_type=jnp.float32,
        ).astype(jnp.bfloat16)
        rdma = pltpu.make_async_remote_copy(
            src_ref=sbuf,
            dst_ref=rbuf,
            send_sem=send_sem,
            recv_sem=recv_sem,
            device_id=_peer(),
            device_id_type=pl.DeviceIdType.MESH,
        )
        rdma.start()
        out_ref[...] = jnp.dot(
            p_ref[pl.ds(my_y * T_LOC, T_LOC), :],
            yb,
            preferred_element_type=jnp.float32,
        )
        rdma.wait()
        out_ref[...] += rbuf[...].astype(jnp.float32)

    return pl.pallas_call(
        body,
        out_shape=jax.ShapeDtypeStruct((T_LOC, D), jnp.float32),
        in_specs=[
            pl.BlockSpec(memory_space=pltpu.VMEM),
            pl.BlockSpec(memory_space=pltpu.VMEM),
        ],
        out_specs=pl.BlockSpec(memory_space=pltpu.VMEM),
        scratch_shapes=[
            pltpu.VMEM((T_LOC, D), jnp.bfloat16),
            pltpu.VMEM((T_LOC, D), jnp.bfloat16),
            pltpu.SemaphoreType.DMA,
            pltpu.SemaphoreType.DMA,
        ],
        compiler_params=pltpu.CompilerParams(collective_id=2),
    )(p, yf)


def kernel(x, router, W1, W2):
    my_y = lax.axis_index("y")

    x_full, r_stack = _ag_x_router(x, router)
    router_full = jnp.moveaxis(r_stack, 0, 1).reshape(D, E_GLB)

    gates_mine = jnp.dot(x, router_full, precision=lax.Precision.HIGHEST)
    gates_full = _ag_gates(gates_mine)

    TT = 2 * T_LOC
    topv, topi = lax.top_k(gates_full, 2)
    w = jax.nn.softmax(topv, axis=-1)
    le = topi - my_y * E_LOC
    valid = (le >= 0) & (le < E_LOC)

    onehot = (
        le[:, :, None] == jnp.arange(E_LOC)[None, None, :]
    ) & valid[:, :, None]
    mask = onehot.any(axis=1)
    pos = jnp.cumsum(mask.astype(jnp.int32), axis=0) - 1

    c_iota = jnp.arange(C, dtype=jnp.int32)[None, :, None]
    q = (
        (jnp.transpose(pos)[:, None, :] == c_iota)
        & jnp.transpose(mask)[:, None, :]
    ).astype(jnp.bfloat16)

    yg = _expert_ffn(x_full, q, W1, W2)

    le_c = jnp.clip(le, 0, E_LOC - 1)
    pos_k = jnp.sum(
        pos[:, None, :] * onehot.astype(jnp.int32), axis=2
    )
    cvalid = valid & (pos_k < C)
    ck = jnp.where(cvalid, le_c * C + pos_k, 0)
    wk = jnp.where(cvalid, w, 0.0)
    slot_iota = jnp.arange(E_LOC * C, dtype=jnp.int32)[None, :]
    p = (
        (ck[:, 0:1] == slot_iota) * wk[:, 0:1]
        + (ck[:, 1:2] == slot_iota) * wk[:, 1:2]
    ).astype(jnp.bfloat16)

    return _combine_reduce_scatter(p, yg.reshape(E_LOC * C, D))


# device time: 190975 ns/iter; 12.0768x vs baseline; 1.0878x over previous
import jax
import jax.numpy as jnp
from jax import lax
from jax.experimental import pallas as pl
from jax.experimental.pallas import tpu as pltpu

T_LOC = 1024
D = 1024
E_LOC = 8
E_GLB = 16
F = 4096
C = 320
FT = 1024


def _peer():
    return (lax.axis_index("x"), 1 - lax.axis_index("y"))


def _peer_barrier():
    bar = pltpu.get_barrier_semaphore()
    pl.semaphore_signal(
        bar, inc=1, device_id=_peer(), device_id_type=pl.DeviceIdType.MESH
    )
    pl.semaphore_wait(bar, 1)


def _ag_x_router(x, router):

    def body(x_ref, r_ref, xf_ref, rs_ref, send_sems, recv_sems):
        my_y = lax.axis_index("y")
        _peer_barrier()
        row0 = my_y * T_LOC
        xf_ref[pl.ds(row0, T_LOC), :] = x_ref[...].astype(jnp.bfloat16)
        rs_ref[my_y] = r_ref[...]
        rdma_x = pltpu.make_async_remote_copy(
            src_ref=xf_ref.at[pl.ds(row0, T_LOC), :],
            dst_ref=xf_ref.at[pl.ds(row0, T_LOC), :],
            send_sem=send_sems.at[0],
            recv_sem=recv_sems.at[0],
            device_id=_peer(),
            device_id_type=pl.DeviceIdType.MESH,
        )
        rdma_r = pltpu.make_async_remote_copy(
            src_ref=rs_ref.at[my_y],
            dst_ref=rs_ref.at[my_y],
            send_sem=send_sems.at[1],
            recv_sem=recv_sems.at[1],
            device_id=_peer(),
            device_id_type=pl.DeviceIdType.MESH,
        )
        rdma_x.start()
        rdma_r.start()
        rdma_r.wait()
        rdma_x.wait()

    return pl.pallas_call(
        body,
        out_shape=(
            jax.ShapeDtypeStruct((2 * T_LOC, D), jnp.bfloat16),
            jax.ShapeDtypeStruct((2, D, E_LOC), jnp.float32),
        ),
        in_specs=[
            pl.BlockSpec(memory_space=pltpu.VMEM),
            pl.BlockSpec(memory_space=pltpu.VMEM),
        ],
        out_specs=(
            pl.BlockSpec(memory_space=pltpu.VMEM),
            pl.BlockSpec(memory_space=pltpu.VMEM),
        ),
        scratch_shapes=[
            pltpu.SemaphoreType.DMA((2,)),
            pltpu.SemaphoreType.DMA((2,)),
        ],
        compiler_params=pltpu.CompilerParams(collective_id=0),
    )(x, router)


def _ag_gates(gates_mine):

    def body(g_ref, gf_ref, send_sem, recv_sem):
        my_y = lax.axis_index("y")
        _peer_barrier()
        row0 = my_y * T_LOC
        gf_ref[pl.ds(row0, T_LOC), :] = g_ref[...]
        rdma = pltpu.make_async_remote_copy(
            src_ref=gf_ref.at[pl.ds(row0, T_LOC), :],
            dst_ref=gf_ref.at[pl.ds(row0, T_LOC), :],
            send_sem=send_sem,
            recv_sem=recv_sem,
            device_id=_peer(),
            device_id_type=pl.DeviceIdType.MESH,
        )
        rdma.start()
        rdma.wait()

    return pl.pallas_call(
        body,
        out_shape=jax.ShapeDtypeStruct((2 * T_LOC, E_GLB), jnp.float32),
        in_specs=[pl.BlockSpec(memory_space=pltpu.VMEM)],
        out_specs=pl.BlockSpec(memory_space=pltpu.VMEM),
        scratch_shapes=[pltpu.SemaphoreType.DMA, pltpu.SemaphoreType.DMA],
        compiler_params=pltpu.CompilerParams(collective_id=1),
    )(gates_mine)


def _expert_ffn(my_x, x_full, q, W1, W2):
    n_fi = (F // 2) // FT

    def body(s_ref, xf_ref, q_ref, w1_ref, w2_ref, out_ref, xg_ref):
        fi = pl.program_id(1)

        @pl.when(fi == 0)
        def _():
            xg_ref[...] = jnp.dot(
                q_ref[0], xf_ref[...], preferred_element_type=jnp.float32
            ).astype(jnp.bfloat16)

        w1 = w1_ref[0].astype(jnp.bfloat16)
        h = jnp.dot(xg_ref[...], w1, preferred_element_type=jnp.float32)
        h = jnp.maximum(h, 0.0).astype(jnp.bfloat16)
        w2 = w2_ref[0].astype(jnp.bfloat16)
        y = jnp.dot(h, w2, preferred_element_type=jnp.float32)

        @pl.when(fi == 0)
        def _():
            out_ref[...] = jnp.zeros_like(out_ref)

        out_ref[...] += y[None]

    grid_spec = pltpu.PrefetchScalarGridSpec(
        num_scalar_prefetch=1,
        grid=(E_LOC, n_fi),
        in_specs=[
            pl.BlockSpec((2 * T_LOC, D), lambda e, fi, s: (0, 0)),
            pl.BlockSpec((1, C, 2 * T_LOC), lambda e, fi, s: (e, 0, 0)),
            pl.BlockSpec((1, D, FT), lambda e, fi, s: (e, 0, fi + s[0] * n_fi)),
            pl.BlockSpec((1, FT, D), lambda e, fi, s: (e, fi + s[0] * n_fi, 0)),
        ],
        out_specs=pl.BlockSpec((1, C, D), lambda e, fi, s: (e, 0, 0)),
        scratch_shapes=[pltpu.VMEM((C, D), jnp.bfloat16)],
    )

    return pl.pallas_call(
        body,
        grid_spec=grid_spec,
        out_shape=jax.ShapeDtypeStruct((E_LOC, C, D), jnp.float32),
        compiler_params=pltpu.CompilerParams(
            dimension_semantics=("arbitrary", "arbitrary")
        ),
    )(my_x, x_full, q, W1, W2)


def _combine_reduce_scatter(p, yf):

    def body(p_ref, yf_ref, out_ref, sbuf, rbuf, sbuf_x, rbuf_x, sems):
        my_y = lax.axis_index("y")
        my_x = lax.axis_index("x")
        peer_y = (my_x, 1 - my_y)
        peer_x = (1 - my_x, my_y)
        bar = pltpu.get_barrier_semaphore()
        for nbr in (peer_y, peer_x):
            pl.semaphore_signal(
                bar, inc=1, device_id=nbr,
                device_id_type=pl.DeviceIdType.MESH,
            )
        pl.semaphore_wait(bar, 2)

        yb = yf_ref[...].astype(jnp.bfloat16)
        sbuf[...] = jnp.dot(
            p_ref[pl.ds((1 - my_y) * T_LOC, T_LOC), :],
            yb,
            preferred_element_type=jnp.float32,
        ).astype(jnp.bfloat16)
        rdma_y = pltpu.make_async_remote_copy(
            src_ref=sbuf,
            dst_ref=rbuf,
            send_sem=sems.at[0],
            recv_sem=sems.at[1],
            device_id=peer_y,
            device_id_type=pl.DeviceIdType.MESH,
        )
        rdma_y.start()
        out_ref[...] = jnp.dot(
            p_ref[pl.ds(my_y * T_LOC, T_LOC), :],
            yb,
            preferred_element_type=jnp.float32,
        )
        rdma_y.wait()
        out_ref[...] += rbuf[...].astype(jnp.float32)

        sbuf_x[...] = out_ref[...].astype(jnp.bfloat16)
        rdma_x = pltpu.make_async_remote_copy(
            src_ref=sbuf_x,
            dst_ref=rbuf_x,
            send_sem=sems.at[2],
            recv_sem=sems.at[3],
            device_id=peer_x,
            device_id_type=pl.DeviceIdType.MESH,
        )
        rdma_x.start()
        rdma_x.wait()
        out_ref[...] += rbuf_x[...].astype(jnp.float32)

    return pl.pallas_call(
        body,
        out_shape=jax.ShapeDtypeStruct((T_LOC, D), jnp.float32),
        in_specs=[
            pl.BlockSpec(memory_space=pltpu.VMEM),
            pl.BlockSpec(memory_space=pltpu.VMEM),
        ],
        out_specs=pl.BlockSpec(memory_space=pltpu.VMEM),
        scratch_shapes=[
            pltpu.VMEM((T_LOC, D), jnp.bfloat16),
            pltpu.VMEM((T_LOC, D), jnp.bfloat16),
            pltpu.VMEM((T_LOC, D), jnp.bfloat16),
            pltpu.VMEM((T_LOC, D), jnp.bfloat16),
            pltpu.SemaphoreType.DMA((4,)),
        ],
        compiler_params=pltpu.CompilerParams(collective_id=2),
    )(p, yf)


def kernel(x, router, W1, W2):
    my_y = lax.axis_index("y")

    x_full, r_stack = _ag_x_router(x, router)
    router_full = jnp.moveaxis(r_stack, 0, 1).reshape(D, E_GLB)

    gates_mine = jnp.dot(x, router_full, precision=lax.Precision.HIGHEST)
    gates_full = _ag_gates(gates_mine)

    TT = 2 * T_LOC
    topv, topi = lax.top_k(gates_full, 2)
    w = jax.nn.softmax(topv, axis=-1)
    le = topi - my_y * E_LOC
    valid = (le >= 0) & (le < E_LOC)

    onehot = (
        le[:, :, None] == jnp.arange(E_LOC)[None, None, :]
    ) & valid[:, :, None]
    mask = onehot.any(axis=1)
    pos = jnp.cumsum(mask.astype(jnp.int32), axis=0) - 1

    c_iota = jnp.arange(C, dtype=jnp.int32)[None, :, None]
    q = (
        (jnp.transpose(pos)[:, None, :] == c_iota)
        & jnp.transpose(mask)[:, None, :]
    ).astype(jnp.bfloat16)

    my_x = lax.axis_index("x").astype(jnp.int32)
    yg = _expert_ffn(my_x[None], x_full, q, W1, W2)

    le_c = jnp.clip(le, 0, E_LOC - 1)
    pos_k = jnp.sum(
        pos[:, None, :] * onehot.astype(jnp.int32), axis=2
    )
    cvalid = valid & (pos_k < C)
    ck = jnp.where(cvalid, le_c * C + pos_k, 0)
    wk = jnp.where(cvalid, w, 0.0)
    slot_iota = jnp.arange(E_LOC * C, dtype=jnp.int32)[None, :]
    p = (
        (ck[:, 0:1] == slot_iota) * wk[:, 0:1]
        + (ck[:, 1:2] == slot_iota) * wk[:, 1:2]
    ).astype(jnp.bfloat16)

    return _combine_reduce_scatter(p, yg.reshape(E_LOC * C, D))


# device time: 171256 ns/iter; 13.4674x vs baseline; 1.1151x over previous
import jax
import jax.numpy as jnp
from jax import lax
from jax.experimental import pallas as pl
from jax.experimental.pallas import tpu as pltpu

T_LOC = 1024
D = 1024
E_LOC = 8
E_GLB = 16
F = 4096
C = 320
FT = 1024


def _peer():
    return (lax.axis_index("x"), 1 - lax.axis_index("y"))


def _peer_barrier():
    bar = pltpu.get_barrier_semaphore()
    pl.semaphore_signal(
        bar, inc=1, device_id=_peer(), device_id_type=pl.DeviceIdType.MESH
    )
    pl.semaphore_wait(bar, 1)


def _ag_x_router(x, router):

    def body(x_ref, r_ref, xf_ref, rs_ref, send_sems, recv_sems):
        my_y = lax.axis_index("y")
        _peer_barrier()
        row0 = my_y * T_LOC
        xf_ref[pl.ds(row0, T_LOC), :] = x_ref[...].astype(jnp.bfloat16)
        rs_ref[my_y] = r_ref[...]
        rdma_x = pltpu.make_async_remote_copy(
            src_ref=xf_ref.at[pl.ds(row0, T_LOC), :],
            dst_ref=xf_ref.at[pl.ds(row0, T_LOC), :],
            send_sem=send_sems.at[0],
            recv_sem=recv_sems.at[0],
            device_id=_peer(),
            device_id_type=pl.DeviceIdType.MESH,
        )
        rdma_r = pltpu.make_async_remote_copy(
            src_ref=rs_ref.at[my_y],
            dst_ref=rs_ref.at[my_y],
            send_sem=send_sems.at[1],
            recv_sem=recv_sems.at[1],
            device_id=_peer(),
            device_id_type=pl.DeviceIdType.MESH,
        )
        rdma_x.start()
        rdma_r.start()
        rdma_r.wait()
        rdma_x.wait()

    return pl.pallas_call(
        body,
        out_shape=(
            jax.ShapeDtypeStruct((2 * T_LOC, D), jnp.bfloat16),
            jax.ShapeDtypeStruct((2, D, E_LOC), jnp.float32),
        ),
        in_specs=[
            pl.BlockSpec(memory_space=pltpu.VMEM),
            pl.BlockSpec(memory_space=pltpu.VMEM),
        ],
        out_specs=(
            pl.BlockSpec(memory_space=pltpu.VMEM),
            pl.BlockSpec(memory_space=pltpu.VMEM),
        ),
        scratch_shapes=[
            pltpu.SemaphoreType.DMA((2,)),
            pltpu.SemaphoreType.DMA((2,)),
        ],
        compiler_params=pltpu.CompilerParams(collective_id=0),
    )(x, router)


def _ag_gates(gates_mine):

    def body(g_ref, gf_ref, send_sem, recv_sem):
        my_y = lax.axis_index("y")
        _peer_barrier()
        row0 = my_y * T_LOC
        gf_ref[pl.ds(row0, T_LOC), :] = g_ref[...]
        rdma = pltpu.make_async_remote_copy(
            src_ref=gf_ref.at[pl.ds(row0, T_LOC), :],
            dst_ref=gf_ref.at[pl.ds(row0, T_LOC), :],
            send_sem=send_sem,
            recv_sem=recv_sem,
            device_id=_peer(),
            device_id_type=pl.DeviceIdType.MESH,
        )
        rdma.start()
        rdma.wait()

    return pl.pallas_call(
        body,
        out_shape=jax.ShapeDtypeStruct((2 * T_LOC, E_GLB), jnp.float32),
        in_specs=[pl.BlockSpec(memory_space=pltpu.VMEM)],
        out_specs=pl.BlockSpec(memory_space=pltpu.VMEM),
        scratch_shapes=[pltpu.SemaphoreType.DMA, pltpu.SemaphoreType.DMA],
        compiler_params=pltpu.CompilerParams(collective_id=1),
    )(gates_mine)


def _expert_ffn(my_x, x_full, q, W1, W2):
    n_fi = (F // 2) // FT

    def body(s_ref, xf_ref, q_ref, w1_ref, w2_ref, out_ref, xg_ref):
        fi = pl.program_id(1)

        @pl.when(fi == 0)
        def _():
            xg_ref[...] = jnp.dot(
                q_ref[0], xf_ref[...], preferred_element_type=jnp.float32
            ).astype(jnp.bfloat16)

        w1 = w1_ref[0].astype(jnp.bfloat16)
        h = jnp.dot(xg_ref[...], w1, preferred_element_type=jnp.float32)
        h = jnp.maximum(h, 0.0).astype(jnp.bfloat16)
        w2 = w2_ref[0].astype(jnp.bfloat16)
        y = jnp.dot(h, w2, preferred_element_type=jnp.float32)

        @pl.when(fi == 0)
        def _():
            out_ref[...] = jnp.zeros_like(out_ref)

        out_ref[...] += y[None]

    grid_spec = pltpu.PrefetchScalarGridSpec(
        num_scalar_prefetch=1,
        grid=(E_LOC, n_fi),
        in_specs=[
            pl.BlockSpec((2 * T_LOC, D), lambda e, fi, s: (0, 0)),
            pl.BlockSpec((1, C, 2 * T_LOC), lambda e, fi, s: (e, 0, 0)),
            pl.BlockSpec((1, D, FT), lambda e, fi, s: (e, 0, fi + s[0] * n_fi)),
            pl.BlockSpec((1, FT, D), lambda e, fi, s: (e, fi + s[0] * n_fi, 0)),
        ],
        out_specs=pl.BlockSpec((1, C, D), lambda e, fi, s: (e, 0, 0)),
        scratch_shapes=[pltpu.VMEM((C, D), jnp.bfloat16)],
    )

    return pl.pallas_call(
        body,
        grid_spec=grid_spec,
        out_shape=jax.ShapeDtypeStruct((E_LOC, C, D), jnp.float32),
        compiler_params=pltpu.CompilerParams(
            dimension_semantics=("arbitrary", "arbitrary")
        ),
    )(my_x, x_full, q, W1, W2)


def _combine_reduce_scatter(p, yf):

    NC = 4
    CH = T_LOC // NC

    def body(p_ref, yf_ref, out_ref, sby, rby, sbx, rbx,
             ysend, yrecv, xsend, xrecv):
        my_y = lax.axis_index("y")
        my_x = lax.axis_index("x")
        peer_y = (my_x, 1 - my_y)
        peer_x = (1 - my_x, my_y)
        bar = pltpu.get_barrier_semaphore()
        for nbr in (peer_y, peer_x):
            pl.semaphore_signal(
                bar, inc=1, device_id=nbr,
                device_id_type=pl.DeviceIdType.MESH,
            )
        pl.semaphore_wait(bar, 2)

        yb = yf_ref[...].astype(jnp.bfloat16)
        prow = (1 - my_y) * T_LOC
        mrow = my_y * T_LOC

        y_rdmas = []
        for i in range(NC):
            v = jnp.dot(
                p_ref[pl.ds(prow + i * CH, CH), :],
                yb,
                preferred_element_type=jnp.float32,
            )
            sby[i] = v.astype(jnp.bfloat16)
            r = pltpu.make_async_remote_copy(
                src_ref=sby.at[i],
                dst_ref=rby.at[i],
                send_sem=ysend.at[i],
                recv_sem=yrecv.at[i],
                device_id=peer_y,
                device_id_type=pl.DeviceIdType.MESH,
            )
            r.start()
            y_rdmas.append(r)

        x_rdmas = []
        for i in range(NC):
            v = jnp.dot(
                p_ref[pl.ds(mrow + i * CH, CH), :],
                yb,
                preferred_element_type=jnp.float32,
            )
            y_rdmas[i].wait_recv()
            v = v + rby[i].astype(jnp.float32)
            out_ref[pl.ds(i * CH, CH), :] = v
            sbx[i] = v.astype(jnp.bfloat16)
            r = pltpu.make_async_remote_copy(
                src_ref=sbx.at[i],
                dst_ref=rbx.at[i],
                send_sem=xsend.at[i],
                recv_sem=xrecv.at[i],
                device_id=peer_x,
                device_id_type=pl.DeviceIdType.MESH,
            )
            r.start()
            x_rdmas.append(r)

        for i in range(NC):
            x_rdmas[i].wait_recv()
            out_ref[pl.ds(i * CH, CH), :] += rbx[i].astype(jnp.float32)
        for r in y_rdmas:
            r.wait_send()
        for r in x_rdmas:
            r.wait_send()

    return pl.pallas_call(
        body,
        out_shape=jax.ShapeDtypeStruct((T_LOC, D), jnp.float32),
        in_specs=[
            pl.BlockSpec(memory_space=pltpu.VMEM),
            pl.BlockSpec(memory_space=pltpu.VMEM),
        ],
        out_specs=pl.BlockSpec(memory_space=pltpu.VMEM),
        scratch_shapes=[
            pltpu.VMEM((NC, CH, D), jnp.bfloat16),
            pltpu.VMEM((NC, CH, D), jnp.bfloat16),
            pltpu.VMEM((NC, CH, D), jnp.bfloat16),
            pltpu.VMEM((NC, CH, D), jnp.bfloat16),
            pltpu.SemaphoreType.DMA((NC,)),
            pltpu.SemaphoreType.DMA((NC,)),
            pltpu.SemaphoreType.DMA((NC,)),
            pltpu.SemaphoreType.DMA((NC,)),
        ],
        compiler_params=pltpu.CompilerParams(collective_id=2),
    )(p, yf)


def kernel(x, router, W1, W2):
    my_y = lax.axis_index("y")

    x_full, r_stack = _ag_x_router(x, router)
    router_full = jnp.moveaxis(r_stack, 0, 1).reshape(D, E_GLB)

    gates_mine = jnp.dot(x, router_full, precision=lax.Precision.HIGHEST)
    gates_full = _ag_gates(gates_mine)

    TT = 2 * T_LOC
    topv, topi = lax.top_k(gates_full, 2)
    w = jax.nn.softmax(topv, axis=-1)
    le = topi - my_y * E_LOC
    valid = (le >= 0) & (le < E_LOC)

    onehot = (
        le[:, :, None] == jnp.arange(E_LOC)[None, None, :]
    ) & valid[:, :, None]
    mask = onehot.any(axis=1)
    pos = jnp.cumsum(mask.astype(jnp.int32), axis=0) - 1

    c_iota = jnp.arange(C, dtype=jnp.int32)[None, :, None]
    q = (
        (jnp.transpose(pos)[:, None, :] == c_iota)
        & jnp.transpose(mask)[:, None, :]
    ).astype(jnp.bfloat16)

    my_x = lax.axis_index("x").astype(jnp.int32)
    yg = _expert_ffn(my_x[None], x_full, q, W1, W2)

    le_c = jnp.clip(le, 0, E_LOC - 1)
    pos_k = jnp.sum(
        pos[:, None, :] * onehot.astype(jnp.int32), axis=2
    )
    cvalid = valid & (pos_k < C)
    ck = jnp.where(cvalid, le_c * C + pos_k, 0)
    wk = jnp.where(cvalid, w, 0.0)
    slot_iota = jnp.arange(E_LOC * C, dtype=jnp.int32)[None, :]
    p = (
        (ck[:, 0:1] == slot_iota) * wk[:, 0:1]
        + (ck[:, 1:2] == slot_iota) * wk[:, 1:2]
    ).astype(jnp.bfloat16)

    return _combine_reduce_scatter(p, yg.reshape(E_LOC * C, D))


# device time: 170198 ns/iter; 13.5511x vs baseline; 1.0062x over previous
import jax
import jax.numpy as jnp
from jax import lax
from jax.experimental import pallas as pl
from jax.experimental.pallas import tpu as pltpu

T_LOC = 1024
D = 1024
E_LOC = 8
E_GLB = 16
F = 4096
C = 320
FT = 1024


def _peer():
    return (lax.axis_index("x"), 1 - lax.axis_index("y"))


def _peer_barrier():
    bar = pltpu.get_barrier_semaphore()
    pl.semaphore_signal(
        bar, inc=1, device_id=_peer(), device_id_type=pl.DeviceIdType.MESH
    )
    pl.semaphore_wait(bar, 1)


def _ag_x_gates(x, router):

    def body(x_ref, r_ref, xf_ref, gf_ref, rs_ref, send_sems, recv_sems):
        my_y = lax.axis_index("y")
        _peer_barrier()
        row0 = my_y * T_LOC
        xf_ref[pl.ds(row0, T_LOC), :] = x_ref[...].astype(jnp.bfloat16)
        rs_ref[my_y] = r_ref[...]
        rdma_x = pltpu.make_async_remote_copy(
            src_ref=xf_ref.at[pl.ds(row0, T_LOC), :],
            dst_ref=xf_ref.at[pl.ds(row0, T_LOC), :],
            send_sem=send_sems.at[0],
            recv_sem=recv_sems.at[0],
            device_id=_peer(),
            device_id_type=pl.DeviceIdType.MESH,
        )
        rdma_r = pltpu.make_async_remote_copy(
            src_ref=rs_ref.at[my_y],
            dst_ref=rs_ref.at[my_y],
            send_sem=send_sems.at[1],
            recv_sem=recv_sems.at[1],
            device_id=_peer(),
            device_id_type=pl.DeviceIdType.MESH,
        )
        rdma_x.start()
        rdma_r.start()
        rdma_r.wait()
        rcat = jnp.concatenate([rs_ref[0], rs_ref[1]], axis=1)
        gf_ref[pl.ds(row0, T_LOC), :] = jax.lax.dot(
            x_ref[...], rcat,
            precision=lax.Precision.HIGHEST,
            preferred_element_type=jnp.float32,
        )
        rdma_g = pltpu.make_async_remote_copy(
            src_ref=gf_ref.at[pl.ds(row0, T_LOC), :],
            dst_ref=gf_ref.at[pl.ds(row0, T_LOC), :],
            send_sem=send_sems.at[2],
            recv_sem=recv_sems.at[2],
            device_id=_peer(),
            device_id_type=pl.DeviceIdType.MESH,
        )
        rdma_g.start()
        rdma_g.wait()
        rdma_x.wait()

    return pl.pallas_call(
        body,
        out_shape=(
            jax.ShapeDtypeStruct((2 * T_LOC, D), jnp.bfloat16),
            jax.ShapeDtypeStruct((2 * T_LOC, E_GLB), jnp.float32),
        ),
        in_specs=[
            pl.BlockSpec(memory_space=pltpu.VMEM),
            pl.BlockSpec(memory_space=pltpu.VMEM),
        ],
        out_specs=(
            pl.BlockSpec(memory_space=pltpu.VMEM),
            pl.BlockSpec(memory_space=pltpu.VMEM),
        ),
        scratch_shapes=[
            pltpu.VMEM((2, D, E_LOC), jnp.float32),
            pltpu.SemaphoreType.DMA((3,)),
            pltpu.SemaphoreType.DMA((3,)),
        ],
        compiler_params=pltpu.CompilerParams(collective_id=0),
    )(x, router)


def _expert_ffn(my_x, x_full, q, W1, W2):
    n_fi = (F // 2) // FT

    def body(s_ref, xf_ref, q_ref, w1_ref, w2_ref, out_ref, xg_ref):
        fi = pl.program_id(1)

        @pl.when(fi == 0)
        def _():
            xg_ref[...] = jnp.dot(
                q_ref[0], xf_ref[...], preferred_element_type=jnp.float32
            ).astype(jnp.bfloat16)

        w1 = w1_ref[0].astype(jnp.bfloat16)
        h = jnp.dot(xg_ref[...], w1, preferred_element_type=jnp.float32)
        h = jnp.maximum(h, 0.0).astype(jnp.bfloat16)
        w2 = w2_ref[0].astype(jnp.bfloat16)
        y = jnp.dot(h, w2, preferred_element_type=jnp.float32)

        @pl.when(fi == 0)
        def _():
            out_ref[...] = jnp.zeros_like(out_ref)

        out_ref[...] += y[None]

    grid_spec = pltpu.PrefetchScalarGridSpec(
        num_scalar_prefetch=1,
        grid=(E_LOC, n_fi),
        in_specs=[
            pl.BlockSpec((2 * T_LOC, D), lambda e, fi, s: (0, 0)),
            pl.BlockSpec((1, C, 2 * T_LOC), lambda e, fi, s: (e, 0, 0)),
            pl.BlockSpec((1, D, FT), lambda e, fi, s: (e, 0, fi + s[0] * n_fi)),
            pl.BlockSpec((1, FT, D), lambda e, fi, s: (e, fi + s[0] * n_fi, 0)),
        ],
        out_specs=pl.BlockSpec((1, C, D), lambda e, fi, s: (e, 0, 0)),
        scratch_shapes=[pltpu.VMEM((C, D), jnp.bfloat16)],
    )

    return pl.pallas_call(
        body,
        grid_spec=grid_spec,
        out_shape=jax.ShapeDtypeStruct((E_LOC, C, D), jnp.float32),
        compiler_params=pltpu.CompilerParams(
            dimension_semantics=("arbitrary", "arbitrary")
        ),
    )(my_x, x_full, q, W1, W2)


def _combine_reduce_scatter(p, yf):

    NC = 4
    CH = T_LOC // NC

    def body(p_ref, yf_ref, out_ref, sby, rby, sbx, rbx,
             ysend, yrecv, xsend, xrecv):
        my_y = lax.axis_index("y")
        my_x = lax.axis_index("x")
        peer_y = (my_x, 1 - my_y)
        peer_x = (1 - my_x, my_y)
        bar = pltpu.get_barrier_semaphore()
        for nbr in (peer_y, peer_x):
            pl.semaphore_signal(
                bar, inc=1, device_id=nbr,
                device_id_type=pl.DeviceIdType.MESH,
            )
        pl.semaphore_wait(bar, 2)

        yb = yf_ref[...].astype(jnp.bfloat16)
        prow = (1 - my_y) * T_LOC
        mrow = my_y * T_LOC

        y_rdmas = []
        for i in range(NC):
            v = jnp.dot(
                p_ref[pl.ds(prow + i * CH, CH), :],
                yb,
                preferred_element_type=jnp.float32,
            )
            sby[i] = v.astype(jnp.bfloat16)
            r = pltpu.make_async_remote_copy(
                src_ref=sby.at[i],
                dst_ref=rby.at[i],
                send_sem=ysend.at[i],
                recv_sem=yrecv.at[i],
                device_id=peer_y,
                device_id_type=pl.DeviceIdType.MESH,
            )
            r.start()
            y_rdmas.append(r)

        x_rdmas = []
        for i in range(NC):
            v = jnp.dot(
                p_ref[pl.ds(mrow + i * CH, CH), :],
                yb,
                preferred_element_type=jnp.float32,
            )
            y_rdmas[i].wait_recv()
            v = v + rby[i].astype(jnp.float32)
            out_ref[pl.ds(i * CH, CH), :] = v
            sbx[i] = v.astype(jnp.bfloat16)
            r = pltpu.make_async_remote_copy(
                src_ref=sbx.at[i],
                dst_ref=rbx.at[i],
                send_sem=xsend.at[i],
                recv_sem=xrecv.at[i],
                device_id=peer_x,
                device_id_type=pl.DeviceIdType.MESH,
            )
            r.start()
            x_rdmas.append(r)

        for i in range(NC):
            x_rdmas[i].wait_recv()
            out_ref[pl.ds(i * CH, CH), :] += rbx[i].astype(jnp.float32)
        for r in y_rdmas:
            r.wait_send()
        for r in x_rdmas:
            r.wait_send()

    return pl.pallas_call(
        body,
        out_shape=jax.ShapeDtypeStruct((T_LOC, D), jnp.float32),
        in_specs=[
            pl.BlockSpec(memory_space=pltpu.VMEM),
            pl.BlockSpec(memory_space=pltpu.VMEM),
        ],
        out_specs=pl.BlockSpec(memory_space=pltpu.VMEM),
        scratch_shapes=[
            pltpu.VMEM((NC, CH, D), jnp.bfloat16),
            pltpu.VMEM((NC, CH, D), jnp.bfloat16),
            pltpu.VMEM((NC, CH, D), jnp.bfloat16),
            pltpu.VMEM((NC, CH, D), jnp.bfloat16),
            pltpu.SemaphoreType.DMA((NC,)),
            pltpu.SemaphoreType.DMA((NC,)),
            pltpu.SemaphoreType.DMA((NC,)),
            pltpu.SemaphoreType.DMA((NC,)),
        ],
        compiler_params=pltpu.CompilerParams(collective_id=2),
    )(p, yf)


def kernel(x, router, W1, W2):
    my_y = lax.axis_index("y")

    x_full, gates_full = _ag_x_gates(x, router)

    TT = 2 * T_LOC
    topv, topi = lax.top_k(gates_full, 2)
    w = jax.nn.softmax(topv, axis=-1)
    le = topi - my_y * E_LOC
    valid = (le >= 0) & (le < E_LOC)

    onehot = (
        le[:, :, None] == jnp.arange(E_LOC)[None, None, :]
    ) & valid[:, :, None]
    mask = onehot.any(axis=1)
    pos = jnp.cumsum(mask.astype(jnp.int32), axis=0) - 1

    c_iota = jnp.arange(C, dtype=jnp.int32)[None, :, None]
    q = (
        (jnp.transpose(pos)[:, None, :] == c_iota)
        & jnp.transpose(mask)[:, None, :]
    ).astype(jnp.bfloat16)

    my_x = lax.axis_index("x").astype(jnp.int32)
    yg = _expert_ffn(my_x[None], x_full, q, W1, W2)

    le_c = jnp.clip(le, 0, E_LOC - 1)
    pos_k = jnp.sum(
        pos[:, None, :] * onehot.astype(jnp.int32), axis=2
    )
    cvalid = valid & (pos_k < C)
    ck = jnp.where(cvalid, le_c * C + pos_k, 0)
    wk = jnp.where(cvalid, w, 0.0)
    slot_iota = jnp.arange(E_LOC * C, dtype=jnp.int32)[None, :]
    p = (
        (ck[:, 0:1] == slot_iota) * wk[:, 0:1]
        + (ck[:, 1:2] == slot_iota) * wk[:, 1:2]
    ).astype(jnp.bfloat16)

    return _combine_reduce_scatter(p, yg.reshape(E_LOC * C, D))


# device time: 165668 ns/iter; 13.9217x vs baseline; 1.0273x over previous
import jax
import jax.numpy as jnp
from jax import lax
from jax.experimental import pallas as pl
from jax.experimental.pallas import tpu as pltpu

T_LOC = 1024
D = 1024
E_LOC = 8
E_GLB = 16
F = 4096
C = 320
FT = 1024


def _peer():
    return (lax.axis_index("x"), 1 - lax.axis_index("y"))


def _peer_barrier():
    bar = pltpu.get_barrier_semaphore()
    pl.semaphore_signal(
        bar, inc=1, device_id=_peer(), device_id_type=pl.DeviceIdType.MESH
    )
    pl.semaphore_wait(bar, 1)


def _ag_x_gates(x, router):

    def body(x_ref, r_ref, xf_ref, gf_ref, rs_ref, send_sems, recv_sems):
        my_y = lax.axis_index("y")
        _peer_barrier()
        row0 = my_y * T_LOC
        xf_ref[pl.ds(row0, T_LOC), :] = x_ref[...].astype(jnp.bfloat16)
        rs_ref[my_y] = r_ref[...]
        rdma_x = pltpu.make_async_remote_copy(
            src_ref=xf_ref.at[pl.ds(row0, T_LOC), :],
            dst_ref=xf_ref.at[pl.ds(row0, T_LOC), :],
            send_sem=send_sems.at[0],
            recv_sem=recv_sems.at[0],
            device_id=_peer(),
            device_id_type=pl.DeviceIdType.MESH,
        )
        rdma_r = pltpu.make_async_remote_copy(
            src_ref=rs_ref.at[my_y],
            dst_ref=rs_ref.at[my_y],
            send_sem=send_sems.at[1],
            recv_sem=recv_sems.at[1],
            device_id=_peer(),
            device_id_type=pl.DeviceIdType.MESH,
        )
        rdma_r.start()
        rdma_x.start()
        rdma_r.wait()
        rcat = jnp.concatenate([rs_ref[0], rs_ref[1]], axis=1)
        gf_ref[pl.ds(row0, T_LOC), :] = jax.lax.dot(
            x_ref[...], rcat,
            precision=lax.Precision.HIGHEST,
            preferred_element_type=jnp.float32,
        )
        rdma_g = pltpu.make_async_remote_copy(
            src_ref=gf_ref.at[pl.ds(row0, T_LOC), :],
            dst_ref=gf_ref.at[pl.ds(row0, T_LOC), :],
            send_sem=send_sems.at[2],
            recv_sem=recv_sems.at[2],
            device_id=_peer(),
            device_id_type=pl.DeviceIdType.MESH,
        )
        rdma_g.start()
        rdma_g.wait()
        rdma_x.wait()

    return pl.pallas_call(
        body,
        out_shape=(
            jax.ShapeDtypeStruct((2 * T_LOC, D), jnp.bfloat16),
            jax.ShapeDtypeStruct((2 * T_LOC, E_GLB), jnp.float32),
        ),
        in_specs=[
            pl.BlockSpec(memory_space=pltpu.VMEM),
            pl.BlockSpec(memory_space=pltpu.VMEM),
        ],
        out_specs=(
            pl.BlockSpec(memory_space=pltpu.VMEM),
            pl.BlockSpec(memory_space=pltpu.VMEM),
        ),
        scratch_shapes=[
            pltpu.VMEM((2, D, E_LOC), jnp.float32),
            pltpu.SemaphoreType.DMA((3,)),
            pltpu.SemaphoreType.DMA((3,)),
        ],
        compiler_params=pltpu.CompilerParams(collective_id=0),
    )(x, router)


def _expert_ffn(my_x, x_full, q, W1, W2):
    n_fi = (F // 2) // FT

    def body(s_ref, xf_ref, q_ref, w1_ref, w2_ref, out_ref, xg_ref):
        fi = pl.program_id(1)

        @pl.when(fi == 0)
        def _():
            xg_ref[...] = jnp.dot(
                q_ref[0], xf_ref[...], preferred_element_type=jnp.float32
            ).astype(jnp.bfloat16)

        w1 = w1_ref[0].astype(jnp.bfloat16)
        h = jnp.dot(xg_ref[...], w1, preferred_element_type=jnp.float32)
        h = jnp.maximum(h, 0.0).astype(jnp.bfloat16)
        w2 = w2_ref[0].astype(jnp.bfloat16)
        y = jnp.dot(h, w2, preferred_element_type=jnp.float32)

        @pl.when(fi == 0)
        def _():
            out_ref[...] = jnp.zeros_like(out_ref)

        out_ref[...] += y[None]

    grid_spec = pltpu.PrefetchScalarGridSpec(
        num_scalar_prefetch=1,
        grid=(E_LOC, n_fi),
        in_specs=[
            pl.BlockSpec((2 * T_LOC, D), lambda e, fi, s: (0, 0)),
            pl.BlockSpec((1, C, 2 * T_LOC), lambda e, fi, s: (e, 0, 0)),
            pl.BlockSpec((1, D, FT), lambda e, fi, s: (e, 0, fi + s[0] * n_fi)),
            pl.BlockSpec((1, FT, D), lambda e, fi, s: (e, fi + s[0] * n_fi, 0)),
        ],
        out_specs=pl.BlockSpec((1, C, D), lambda e, fi, s: (e, 0, 0)),
        scratch_shapes=[pltpu.VMEM((C, D), jnp.bfloat16)],
    )

    return pl.pallas_call(
        body,
        grid_spec=grid_spec,
        out_shape=jax.ShapeDtypeStruct((E_LOC, C, D), jnp.float32),
        compiler_params=pltpu.CompilerParams(
            dimension_semantics=("arbitrary", "arbitrary")
        ),
    )(my_x, x_full, q, W1, W2)


def _combine_reduce_scatter(p, yf):

    NC = 4
    CH = T_LOC // NC

    def body(p_ref, yf_ref, out_ref, sby, rby, sbx, rbx,
             ysend, yrecv, xsend, xrecv):
        my_y = lax.axis_index("y")
        my_x = lax.axis_index("x")
        peer_y = (my_x, 1 - my_y)
        peer_x = (1 - my_x, my_y)
        bar = pltpu.get_barrier_semaphore()
        for nbr in (peer_y, peer_x):
            pl.semaphore_signal(
                bar, inc=1, device_id=nbr,
                device_id_type=pl.DeviceIdType.MESH,
            )
        pl.semaphore_wait(bar, 2)

        yb = yf_ref[...].astype(jnp.bfloat16)
        prow = (1 - my_y) * T_LOC
        mrow = my_y * T_LOC

        y_rdmas = []
        for i in range(NC):
            v = jnp.dot(
                p_ref[pl.ds(prow + i * CH, CH), :],
                yb,
                preferred_element_type=jnp.float32,
            )
            sby[i] = v.astype(jnp.bfloat16)
            r = pltpu.make_async_remote_copy(
                src_ref=sby.at[i],
                dst_ref=rby.at[i],
                send_sem=ysend.at[i],
                recv_sem=yrecv.at[i],
                device_id=peer_y,
                device_id_type=pl.DeviceIdType.MESH,
            )
            r.start()
            y_rdmas.append(r)

        x_rdmas = []
        for i in range(NC):
            v = jnp.dot(
                p_ref[pl.ds(mrow + i * CH, CH), :],
                yb,
                preferred_element_type=jnp.float32,
            )
            y_rdmas[i].wait_recv()
            v = v + rby[i].astype(jnp.float32)
            out_ref[pl.ds(i * CH, CH), :] = v
            sbx[i] = v.astype(jnp.bfloat16)
            r = pltpu.make_async_remote_copy(
                src_ref=sbx.at[i],
                dst_ref=rbx.at[i],
                send_sem=xsend.at[i],
                recv_sem=xrecv.at[i],
                device_id=peer_x,
                device_id_type=pl.DeviceIdType.MESH,
            )
            r.start()
            x_rdmas.append(r)

        for i in range(NC):
            x_rdmas[i].wait_recv()
            out_ref[pl.ds(i * CH, CH), :] += rbx[i].astype(jnp.float32)
        for r in y_rdmas:
            r.wait_send()
        for r in x_rdmas:
            r.wait_send()

    return pl.pallas_call(
        body,
        out_shape=jax.ShapeDtypeStruct((T_LOC, D), jnp.float32),
        in_specs=[
            pl.BlockSpec(memory_space=pltpu.VMEM),
            pl.BlockSpec(memory_space=pltpu.VMEM),
        ],
        out_specs=pl.BlockSpec(memory_space=pltpu.VMEM),
        scratch_shapes=[
            pltpu.VMEM((NC, CH, D), jnp.bfloat16),
            pltpu.VMEM((NC, CH, D), jnp.bfloat16),
            pltpu.VMEM((NC, CH, D), jnp.bfloat16),
            pltpu.VMEM((NC, CH, D), jnp.bfloat16),
            pltpu.SemaphoreType.DMA((NC,)),
            pltpu.SemaphoreType.DMA((NC,)),
            pltpu.SemaphoreType.DMA((NC,)),
            pltpu.SemaphoreType.DMA((NC,)),
        ],
        compiler_params=pltpu.CompilerParams(collective_id=2),
    )(p, yf)


def kernel(x, router, W1, W2):
    my_y = lax.axis_index("y")

    x_full, gates_full = _ag_x_gates(x, router)

    TT = 2 * T_LOC
    topv, topi = lax.top_k(gates_full, 2)
    w = jax.nn.softmax(topv, axis=-1)
    le = topi - my_y * E_LOC
    valid = (le >= 0) & (le < E_LOC)

    onehot = (
        le[:, :, None] == jnp.arange(E_LOC)[None, None, :]
    ) & valid[:, :, None]
    mask = onehot.any(axis=1)
    pos = jnp.cumsum(mask.astype(jnp.int32), axis=0) - 1

    c_iota = jnp.arange(C, dtype=jnp.int32)[None, :, None]
    q = (
        (jnp.transpose(pos)[:, None, :] == c_iota)
        & jnp.transpose(mask)[:, None, :]
    ).astype(jnp.bfloat16)

    my_x = lax.axis_index("x").astype(jnp.int32)
    yg = _expert_ffn(my_x[None], x_full, q, W1, W2)

    le_c = jnp.clip(le, 0, E_LOC - 1)
    pos_k = jnp.sum(
        pos[:, None, :] * onehot.astype(jnp.int32), axis=2
    )
    cvalid = valid & (pos_k < C)
    ck = jnp.where(cvalid, le_c * C + pos_k, 0)
    wk = jnp.where(cvalid, w, 0.0)
    slot_iota = jnp.arange(E_LOC * C, dtype=jnp.int32)[None, :]
    p = (
        (ck[:, 0:1] == slot_iota) * wk[:, 0:1]
        + (ck[:, 1:2] == slot_iota) * wk[:, 1:2]
    ).astype(jnp.bfloat16)

    return _combine_reduce_scatter(p, yg.reshape(E_LOC * C, D))
